# baseline (speedup 1.0000x reference)
import dataclasses
import functools

import jax
import jax.numpy as jnp
from jax import lax
from jax.experimental import pallas as pl
from jax.experimental.pallas import tpu as pltpu
from jax.experimental.pallas import tpu_sc as plsc

N = 10000
E = 320000
D = 128
H = 256
O = 64

NLANE = 16
NCORE = 2
NSUB = 16
NW = NCORE * NSUB
C = 64
K = 164
KA = 108
KB = 2 * K - KA
EPAD = NW * K * C
TOT_CHUNKS = EPAD // C
NACC = 10240
ROWS_PER_TILE = NACC // NSUB
PADN = 10016
DUMMY = N


def _tc_head(x, W, att_src, att_dst):
    def body(x_ref, w_ref, as_ref, ad_ref, h_ref, s_ref, d_ref, c_ref):
        h = jnp.dot(x_ref[...], w_ref[...], preferred_element_type=jnp.float32)
        h_ref[...] = h
        a_s = jnp.sum(h * as_ref[...][None, :], axis=1, keepdims=True)
        a_d = jnp.sum(h * ad_ref[...][None, :], axis=1, keepdims=True)
        s_ref[...] = a_s
        d_ref[...] = a_d
        c = jnp.maximum(jnp.max(a_s) + jnp.max(a_d), 0.0)
        c_ref[...] = jnp.full((1, NLANE), c, jnp.float32)

    return pl.pallas_call(
        body,
        out_shape=(
            jax.ShapeDtypeStruct((N, D), jnp.float32),
            jax.ShapeDtypeStruct((N, 1), jnp.float32),
            jax.ShapeDtypeStruct((N, 1), jnp.float32),
            jax.ShapeDtypeStruct((1, NLANE), jnp.float32),
        ),
    )(x, W, att_src, att_dst)


def _sc_gat(h, src_idx, dst_idx, a_s, a_d, cvec):
    mesh = plsc.VectorSubcoreMesh(core_axis_name="c", subcore_axis_name="s")
    cp = pltpu.CompilerParams()
    if "needs_layout_passes" in pltpu.CompilerParams.__dataclass_fields__:
        cp = dataclasses.replace(cp, needs_layout_passes=False)

    @functools.partial(
        pl.kernel,
        compiler_params=cp,
        out_type=(
            jax.ShapeDtypeStruct((NCORE, NACC, D), jnp.float32),
            jax.ShapeDtypeStruct((NW, NACC), jnp.float32),
        ),
        mesh=mesh,
        scratch_types=[
            pltpu.VMEM((PADN,), jnp.float32),
            pltpu.VMEM((PADN,), jnp.float32),
            pltpu.VMEM((NLANE,), jnp.float32),
            pltpu.VMEM((C,), jnp.int32),
            pltpu.VMEM((C,), jnp.int32),
            pltpu.VMEM((C,), jnp.int32),
            pltpu.VMEM((C,), jnp.int32),
            pltpu.VMEM((C,), jnp.int32),
            pltpu.VMEM((C,), jnp.int32),
            pltpu.VMEM((C, D), jnp.float32),
            pltpu.VMEM((C, D), jnp.float32),
            pltpu.VMEM((C,), jnp.float32),
            pltpu.VMEM((C,), jnp.float32),
            pltpu.VMEM((NACC,), jnp.float32),
            pltpu.VMEM_SHARED((NACC, D), jnp.float32),
            pltpu.SemaphoreType.DMA,
            pltpu.SemaphoreType.DMA,
            pltpu.SemaphoreType.DMA,
            pltpu.SemaphoreType.DMA,
            pltpu.SemaphoreType.DMA,
            pltpu.SemaphoreType.DMA,
        ],
    )
    def kern(h_hbm, src_hbm, dst_hbm, as_hbm, ad_hbm, c_hbm, acc_out, den_out,
             as_v, ad_v, c_v, srcA, dstA, dsA, srcB, dstB, dsB,
             rowsA, rowsB, pA, pB, den_v, acc_sh,
             semiA, semiB, semgA, semgB, semsA, semsB):
        cid = lax.axis_index("c")
        sid = lax.axis_index("s")
        wid = sid * NCORE + cid

        pltpu.sync_copy(as_hbm, as_v)
        pltpu.sync_copy(ad_hbm, ad_v)
        pltpu.sync_copy(c_hbm, c_v)
        shift = c_v[...]

        slots = ((srcA, dstA, dsA, rowsA, pA, semiA, semgA, semsA),
                 (srcB, dstB, dsB, rowsB, pB, semiB, semgB, semsB))

        def issue_idx(slot, ck):
            src_v, dst_v = slots[slot][0], slots[slot][1]
            semi = slots[slot][5]
            pltpu.async_copy(src_hbm.at[ck], src_v, semi)
            pltpu.async_copy(dst_hbm.at[ck], dst_v, semi)

        def wait_idx(slot):
            src_v, dst_v = slots[slot][0], slots[slot][1]
            semi = slots[slot][5]
            pltpu.make_async_copy(src_hbm.at[0], src_v, semi).wait()
            pltpu.make_async_copy(dst_hbm.at[0], dst_v, semi).wait()

        def issue_gather(slot):
            src_v, rows_v, semg = slots[slot][0], slots[slot][3], slots[slot][6]
            pltpu.async_copy(h_hbm.at[src_v], rows_v, semg)

        def wait_gather(slot):
            rows_v, semg = slots[slot][3], slots[slot][6]
            pltpu.make_async_copy(h_hbm.at[pl.ds(0, C)], rows_v, semg).wait()

        def issue_scatter(slot):
            ds_v, rows_v, sems = slots[slot][2], slots[slot][3], slots[slot][7]
            pltpu.async_copy(rows_v, acc_sh.at[ds_v], sems, add=True)

        def wait_scatter(slot):
            rows_v, sems = slots[slot][3], slots[slot][7]
            pltpu.make_async_copy(rows_v, acc_sh.at[pl.ds(0, C)], sems).wait()

        def compute_p(slot):
            src_v, dst_v, ds_v, _, p_v = slots[slot][:5]
            for b in range(C // NLANE):
                sl = pl.ds(b * NLANE, NLANE)
                d16 = dst_v[sl]
                ds_v[sl] = d16
                av = plsc.load_gather(as_v, [src_v[sl]])
                bv = plsc.load_gather(ad_v, [d16])
                e = av + bv
                e = jnp.maximum(e, e * 0.2)
                p16 = jnp.exp(e - shift)
                p_v[sl] = p16
                plsc.addupdate_scatter(den_v, [d16], p16)

        def scale(slot):
            rows_v, p_v = slots[slot][3], slots[slot][4]

            @pl.loop(0, C)
            def _(j):
                jv = jnp.full((NLANE,), j, jnp.int32)
                pvec = plsc.load_gather(p_v, [jv])
                for q in range(D // NLANE):
                    sl = pl.ds(q * NLANE, NLANE)
                    rows_v[j, sl] = rows_v[j, sl] * pvec

        zv = jnp.zeros((NLANE,), jnp.float32)

        @pl.loop(0, NACC, step=NLANE)
        def _(i):
            den_v[pl.ds(i, NLANE)] = zv

        @pl.loop(0, C)
        def _(j):
            for q in range(D // NLANE):
                rowsA[j, pl.ds(q * NLANE, NLANE)] = zv

        zbase = sid * ROWS_PER_TILE
        for t in range(ROWS_PER_TILE // C):
            pltpu.sync_copy(rowsA, acc_sh.at[pl.ds(zbase + t * C, C)])

        kc = jnp.where(cid == 0, KA, KB)
        chunk0 = jnp.where(cid == 0, sid * KA, NSUB * KA + sid * KB)
        issue_idx(0, chunk0)
        issue_idx(1, chunk0 + 1)
        wait_idx(0)
        issue_gather(0)
        wait_idx(1)
        issue_gather(1)
        plsc.subcore_barrier()

        @pl.loop(0, kc // 2 - 1)
        def _(i):
            k0 = chunk0 + 2 * i
            for s in (0, 1):
                compute_p(s)
                wait_gather(s)
                issue_idx(s, k0 + s + 2)
                scale(s)
                issue_scatter(s)
            for s in (0, 1):
                wait_scatter(s)
                wait_idx(s)
                issue_gather(s)

        for s in (0, 1):
            compute_p(s)
            wait_gather(s)
            scale(s)
            issue_scatter(s)
        wait_scatter(0)
        wait_scatter(1)

        plsc.subcore_barrier()
        pltpu.sync_copy(acc_sh.at[pl.ds(zbase, ROWS_PER_TILE)],
                        acc_out.at[cid, pl.ds(zbase, ROWS_PER_TILE)])
        pltpu.sync_copy(den_v, den_out.at[wid])

    return kern(h, src_idx, dst_idx, a_s, a_d, cvec)


def _tc_mlp(acc, den, bias, W1, b1, W2, b2):
    BR = 1024

    def body(acc_ref, den_ref, bias_ref, w1_ref, b1_ref, w2_ref, b2_ref,
             y_ref):
        g = acc_ref[0] + acc_ref[1]
        d = jnp.sum(den_ref[...], axis=0).reshape(BR, 1)
        gat = g / d + bias_ref[...][None, :]
        z = jnp.dot(gat, w1_ref[...], preferred_element_type=jnp.float32)
        z = jnp.maximum(z + b1_ref[...][None, :], 0.0)
        y = jnp.dot(z, w2_ref[...], preferred_element_type=jnp.float32)
        y_ref[...] = jax.nn.sigmoid(y + b2_ref[...][None, :])

    return pl.pallas_call(
        body,
        grid=(NACC // BR,),
        in_specs=[
            pl.BlockSpec((NCORE, BR, D), lambda i: (0, i, 0)),
            pl.BlockSpec((NW, BR), lambda i: (0, i)),
            pl.BlockSpec((D,), lambda i: (0,)),
            pl.BlockSpec((D, H), lambda i: (0, 0)),
            pl.BlockSpec((H,), lambda i: (0,)),
            pl.BlockSpec((H, O), lambda i: (0, 0)),
            pl.BlockSpec((O,), lambda i: (0,)),
        ],
        out_specs=pl.BlockSpec((BR, O), lambda i: (i, 0)),
        out_shape=jax.ShapeDtypeStruct((NACC, O), jnp.float32),
    )(acc, den, bias, W1, b1, W2, b2)


def kernel(x, edge_index, W, att_src, att_dst, bias, W1, b1, W2, b2):
    h, a_s2, a_d2, cvec = _tc_head(x, W, att_src, att_dst)

    loop = jnp.arange(N, dtype=jnp.int32)
    npad = EPAD - E - N
    src = jnp.concatenate([edge_index[0], loop,
                           jnp.zeros((npad,), jnp.int32)])
    dst = jnp.concatenate([edge_index[1], loop,
                           jnp.full((npad,), DUMMY, jnp.int32)])
    src = src.reshape(TOT_CHUNKS, C)
    dst = dst.reshape(TOT_CHUNKS, C)

    a_s = jnp.pad(a_s2.reshape(-1), (0, PADN - N))
    a_d = jnp.pad(a_d2.reshape(-1), (0, PADN - N))

    acc, den = _sc_gat(h, src, dst, a_s, a_d, cvec.reshape(-1))
    y = _tc_mlp(acc, den, bias, W1, b1, W2, b2)
    return y[:N]


# speedup vs baseline: 1.2602x; 1.2602x over previous
import dataclasses
import functools

import jax
import jax.numpy as jnp
from jax import lax
from jax.experimental import pallas as pl
from jax.experimental.pallas import tpu as pltpu
from jax.experimental.pallas import tpu_sc as plsc

N = 10000
E = 320000
D = 128
H = 256
O = 64

NLANE = 16
NCORE = 2
NSUB = 16
NW = NCORE * NSUB
C = 64
K = 164
KA = 220
KB = 2 * K - KA
EPAD = NW * K * C
TOT_CHUNKS = EPAD // C
NACC = 10240
ROWS_PER_TILE = NACC // NSUB
PADN = 10016
DUMMY = N


def _tc_head(x, W, att_src, att_dst):
    def body(x_ref, w_ref, as_ref, ad_ref, h_ref, s_ref, d_ref, c_ref):
        h = jnp.dot(x_ref[...], w_ref[...], preferred_element_type=jnp.float32)
        h_ref[...] = h
        a_s = jnp.sum(h * as_ref[...][None, :], axis=1, keepdims=True)
        a_d = jnp.sum(h * ad_ref[...][None, :], axis=1, keepdims=True)
        s_ref[...] = a_s
        d_ref[...] = a_d
        c = jnp.maximum(jnp.max(a_s) + jnp.max(a_d), 0.0)
        c_ref[...] = jnp.full((1, NLANE), c, jnp.float32)

    return pl.pallas_call(
        body,
        out_shape=(
            jax.ShapeDtypeStruct((N, D), jnp.float32),
            jax.ShapeDtypeStruct((N, 1), jnp.float32),
            jax.ShapeDtypeStruct((N, 1), jnp.float32),
            jax.ShapeDtypeStruct((1, NLANE), jnp.float32),
        ),
    )(x, W, att_src, att_dst)


def _sc_gat(h, src_idx, dst_idx, a_s, a_d, cvec):
    mesh = plsc.VectorSubcoreMesh(core_axis_name="c", subcore_axis_name="s")
    cp = pltpu.CompilerParams()
    if "needs_layout_passes" in pltpu.CompilerParams.__dataclass_fields__:
        cp = dataclasses.replace(cp, needs_layout_passes=False)

    @functools.partial(
        pl.kernel,
        compiler_params=cp,
        out_type=(
            jax.ShapeDtypeStruct((NCORE, NACC, D), jnp.float32),
            jax.ShapeDtypeStruct((NW, NACC), jnp.float32),
        ),
        mesh=mesh,
        scratch_types=[
            pltpu.VMEM((PADN,), jnp.float32),
            pltpu.VMEM((PADN,), jnp.float32),
            pltpu.VMEM((NLANE,), jnp.float32),
            pltpu.VMEM((C,), jnp.int32),
            pltpu.VMEM((C,), jnp.int32),
            pltpu.VMEM((C,), jnp.int32),
            pltpu.VMEM((C,), jnp.int32),
            pltpu.VMEM((C,), jnp.int32),
            pltpu.VMEM((C,), jnp.int32),
            pltpu.VMEM((C, D), jnp.float32),
            pltpu.VMEM((C, D), jnp.float32),
            pltpu.VMEM((C,), jnp.float32),
            pltpu.VMEM((C,), jnp.float32),
            pltpu.VMEM((NACC,), jnp.float32),
            pltpu.VMEM_SHARED((NACC, D), jnp.float32),
            pltpu.SemaphoreType.DMA,
            pltpu.SemaphoreType.DMA,
            pltpu.SemaphoreType.DMA,
            pltpu.SemaphoreType.DMA,
            pltpu.SemaphoreType.DMA,
            pltpu.SemaphoreType.DMA,
        ],
    )
    def kern(h_hbm, src_hbm, dst_hbm, as_hbm, ad_hbm, c_hbm, acc_out, den_out,
             as_v, ad_v, c_v, srcA, dstA, dsA, srcB, dstB, dsB,
             rowsA, rowsB, pA, pB, den_v, acc_sh,
             semiA, semiB, semgA, semgB, semsA, semsB):
        cid = lax.axis_index("c")
        sid = lax.axis_index("s")
        wid = sid * NCORE + cid

        pltpu.sync_copy(as_hbm, as_v)
        pltpu.sync_copy(ad_hbm, ad_v)
        pltpu.sync_copy(c_hbm, c_v)
        shift = c_v[...]

        slots = ((srcA, dstA, dsA, rowsA, pA, semiA, semgA, semsA),
                 (srcB, dstB, dsB, rowsB, pB, semiB, semgB, semsB))

        def issue_idx(slot, ck):
            src_v, dst_v = slots[slot][0], slots[slot][1]
            semi = slots[slot][5]
            pltpu.async_copy(src_hbm.at[ck], src_v, semi)
            pltpu.async_copy(dst_hbm.at[ck], dst_v, semi)

        def wait_idx(slot):
            src_v, dst_v = slots[slot][0], slots[slot][1]
            semi = slots[slot][5]
            pltpu.make_async_copy(src_hbm.at[0], src_v, semi).wait()
            pltpu.make_async_copy(dst_hbm.at[0], dst_v, semi).wait()

        def issue_gather(slot):
            src_v, rows_v, semg = slots[slot][0], slots[slot][3], slots[slot][6]
            pltpu.async_copy(h_hbm.at[src_v], rows_v, semg)

        def wait_gather(slot):
            rows_v, semg = slots[slot][3], slots[slot][6]
            pltpu.make_async_copy(h_hbm.at[pl.ds(0, C)], rows_v, semg).wait()

        def issue_scatter(slot):
            ds_v, rows_v, sems = slots[slot][2], slots[slot][3], slots[slot][7]
            pltpu.async_copy(rows_v, acc_sh.at[ds_v], sems, add=True)

        def wait_scatter(slot):
            rows_v, sems = slots[slot][3], slots[slot][7]
            pltpu.make_async_copy(rows_v, acc_sh.at[pl.ds(0, C)], sems).wait()

        def compute_p(slot):
            src_v, dst_v, ds_v, _, p_v = slots[slot][:5]
            for b in range(C // NLANE):
                sl = pl.ds(b * NLANE, NLANE)
                d16 = dst_v[sl]
                ds_v[sl] = d16
                av = plsc.load_gather(as_v, [src_v[sl]])
                bv = plsc.load_gather(ad_v, [d16])
                e = av + bv
                e = jnp.maximum(e, e * 0.2)
                p16 = jnp.exp(e - shift)
                p_v[sl] = p16
                plsc.addupdate_scatter(den_v, [d16], p16)

        def scale(slot):
            rows_v, p_v = slots[slot][3], slots[slot][4]

            @pl.loop(0, C)
            def _(j):
                jv = jnp.full((NLANE,), j, jnp.int32)
                pvec = plsc.load_gather(p_v, [jv])
                for q in range(D // NLANE):
                    sl = pl.ds(q * NLANE, NLANE)
                    rows_v[j, sl] = rows_v[j, sl] * pvec

        zv = jnp.zeros((NLANE,), jnp.float32)

        @pl.loop(0, NACC, step=NLANE)
        def _(i):
            den_v[pl.ds(i, NLANE)] = zv

        @pl.loop(0, C)
        def _(j):
            for q in range(D // NLANE):
                rowsA[j, pl.ds(q * NLANE, NLANE)] = zv

        zbase = sid * ROWS_PER_TILE
        for t in range(ROWS_PER_TILE // C):
            pltpu.sync_copy(rowsA, acc_sh.at[pl.ds(zbase + t * C, C)])

        kc = jnp.where(cid == 0, KA, KB)
        chunk0 = jnp.where(cid == 0, sid * KA, NSUB * KA + sid * KB)
        issue_idx(0, chunk0)
        issue_idx(1, chunk0 + 1)
        wait_idx(0)
        issue_gather(0)
        wait_idx(1)
        issue_gather(1)
        plsc.subcore_barrier()

        @pl.loop(0, kc // 2 - 1)
        def _(i):
            k0 = chunk0 + 2 * i
            for s in (0, 1):
                compute_p(s)
                wait_gather(s)
                issue_idx(s, k0 + s + 2)
                scale(s)
                issue_scatter(s)
            for s in (0, 1):
                wait_scatter(s)
                wait_idx(s)
                issue_gather(s)

        for s in (0, 1):
            compute_p(s)
            wait_gather(s)
            scale(s)
            issue_scatter(s)
        wait_scatter(0)
        wait_scatter(1)

        plsc.subcore_barrier()
        pltpu.sync_copy(acc_sh.at[pl.ds(zbase, ROWS_PER_TILE)],
                        acc_out.at[cid, pl.ds(zbase, ROWS_PER_TILE)])
        pltpu.sync_copy(den_v, den_out.at[wid])

    return kern(h, src_idx, dst_idx, a_s, a_d, cvec)


def _tc_mlp(acc, den, bias, W1, b1, W2, b2):
    BR = 1024

    def body(acc_ref, den_ref, bias_ref, w1_ref, b1_ref, w2_ref, b2_ref,
             y_ref):
        g = acc_ref[0] + acc_ref[1]
        d = jnp.sum(den_ref[...], axis=0).reshape(BR, 1)
        gat = g / d + bias_ref[...][None, :]
        z = jnp.dot(gat, w1_ref[...], preferred_element_type=jnp.float32)
        z = jnp.maximum(z + b1_ref[...][None, :], 0.0)
        y = jnp.dot(z, w2_ref[...], preferred_element_type=jnp.float32)
        y_ref[...] = jax.nn.sigmoid(y + b2_ref[...][None, :])

    return pl.pallas_call(
        body,
        grid=(NACC // BR,),
        in_specs=[
            pl.BlockSpec((NCORE, BR, D), lambda i: (0, i, 0)),
            pl.BlockSpec((NW, BR), lambda i: (0, i)),
            pl.BlockSpec((D,), lambda i: (0,)),
            pl.BlockSpec((D, H), lambda i: (0, 0)),
            pl.BlockSpec((H,), lambda i: (0,)),
            pl.BlockSpec((H, O), lambda i: (0, 0)),
            pl.BlockSpec((O,), lambda i: (0,)),
        ],
        out_specs=pl.BlockSpec((BR, O), lambda i: (i, 0)),
        out_shape=jax.ShapeDtypeStruct((NACC, O), jnp.float32),
    )(acc, den, bias, W1, b1, W2, b2)


def kernel(x, edge_index, W, att_src, att_dst, bias, W1, b1, W2, b2):
    h, a_s2, a_d2, cvec = _tc_head(x, W, att_src, att_dst)

    loop = jnp.arange(N, dtype=jnp.int32)
    npad = EPAD - E - N
    src = jnp.concatenate([edge_index[0], loop,
                           jnp.zeros((npad,), jnp.int32)])
    dst = jnp.concatenate([edge_index[1], loop,
                           jnp.full((npad,), DUMMY, jnp.int32)])
    src = src.reshape(TOT_CHUNKS, C)
    dst = dst.reshape(TOT_CHUNKS, C)

    a_s = jnp.pad(a_s2.reshape(-1), (0, PADN - N))
    a_d = jnp.pad(a_d2.reshape(-1), (0, PADN - N))

    acc, den = _sc_gat(h, src, dst, a_s, a_d, cvec.reshape(-1))
    y = _tc_mlp(acc, den, bias, W1, b1, W2, b2)
    return y[:N]


# speedup vs baseline: 1.6246x; 1.2891x over previous
import dataclasses
import functools

import jax
import jax.numpy as jnp
from jax import lax
from jax.experimental import pallas as pl
from jax.experimental.pallas import tpu as pltpu
from jax.experimental.pallas import tpu_sc as plsc

N = 10000
E = 320000
D = 128
H = 256
O = 64

NLANE = 16
NCORE = 2
NSUB = 16
NW = NCORE * NSUB
NSLOT = 4
C = 32
KA = 324
KB = 324
TOT_CHUNKS = NSUB * (KA + KB)
EPAD = TOT_CHUNKS * C
NACC = 10112
ROWS_PER_TILE = NACC // NSUB
PADN = 10016
DUMMY = N


def _tc_head(x, W, att_src, att_dst):
    def body(x_ref, w_ref, as_ref, ad_ref, h_ref, s_ref, d_ref, c_ref):
        h = jnp.dot(x_ref[...], w_ref[...], preferred_element_type=jnp.float32)
        h_ref[...] = h
        a_s = jnp.sum(h * as_ref[...][None, :], axis=1, keepdims=True)
        a_d = jnp.sum(h * ad_ref[...][None, :], axis=1, keepdims=True)
        s_ref[...] = a_s
        d_ref[...] = a_d
        c = jnp.maximum(jnp.max(a_s) + jnp.max(a_d), 0.0)
        c_ref[...] = jnp.full((1, NLANE), c, jnp.float32)

    return pl.pallas_call(
        body,
        out_shape=(
            jax.ShapeDtypeStruct((N, D), jnp.float32),
            jax.ShapeDtypeStruct((N, 1), jnp.float32),
            jax.ShapeDtypeStruct((N, 1), jnp.float32),
            jax.ShapeDtypeStruct((1, NLANE), jnp.float32),
        ),
    )(x, W, att_src, att_dst)


def _sc_gat(h, src_idx, dst_idx, a_s, a_d, cvec):
    mesh = plsc.VectorSubcoreMesh(core_axis_name="c", subcore_axis_name="s")
    cp = pltpu.CompilerParams()
    if "needs_layout_passes" in pltpu.CompilerParams.__dataclass_fields__:
        cp = dataclasses.replace(cp, needs_layout_passes=False)

    scratch = [
        pltpu.VMEM((PADN,), jnp.float32),
        pltpu.VMEM((PADN,), jnp.float32),
        pltpu.VMEM((NLANE,), jnp.float32),
    ]
    for _ in range(NSLOT):
        scratch += [
            pltpu.VMEM((C,), jnp.int32),
            pltpu.VMEM((C,), jnp.int32),
            pltpu.VMEM((C,), jnp.int32),
            pltpu.VMEM((C,), jnp.float32),
            pltpu.VMEM((C, D), jnp.float32),
        ]
    scratch += [
        pltpu.VMEM((NACC,), jnp.float32),
        pltpu.VMEM_SHARED((NACC, D), jnp.float32),
    ]
    scratch += [pltpu.SemaphoreType.DMA] * (3 * NSLOT)

    @functools.partial(
        pl.kernel,
        compiler_params=cp,
        out_type=(
            jax.ShapeDtypeStruct((NCORE, NACC, D), jnp.float32),
            jax.ShapeDtypeStruct((NW, NACC), jnp.float32),
        ),
        mesh=mesh,
        scratch_types=scratch,
    )
    def kern(h_hbm, src_hbm, dst_hbm, as_hbm, ad_hbm, c_hbm, acc_out, den_out,
             *scr):
        as_v, ad_v, c_v = scr[0], scr[1], scr[2]
        slots = [scr[3 + 5 * s: 3 + 5 * (s + 1)] for s in range(NSLOT)]
        den_v = scr[3 + 5 * NSLOT]
        acc_sh = scr[4 + 5 * NSLOT]
        sems = scr[5 + 5 * NSLOT:]
        semi = sems[:NSLOT]
        semg = sems[NSLOT:2 * NSLOT]
        semsc = sems[2 * NSLOT:]

        cid = lax.axis_index("c")
        sid = lax.axis_index("s")

        pltpu.sync_copy(as_hbm, as_v)
        pltpu.sync_copy(ad_hbm, ad_v)
        pltpu.sync_copy(c_hbm, c_v)
        shift = c_v[...]

        def issue_idx(s, ck):
            src_v, dst_v = slots[s][0], slots[s][1]
            pltpu.async_copy(src_hbm.at[ck], src_v, semi[s])
            pltpu.async_copy(dst_hbm.at[ck], dst_v, semi[s])

        def wait_idx(s):
            src_v, dst_v = slots[s][0], slots[s][1]
            pltpu.make_async_copy(src_hbm.at[0], src_v, semi[s]).wait()
            pltpu.make_async_copy(dst_hbm.at[0], dst_v, semi[s]).wait()

        def issue_gather(s):
            pltpu.async_copy(h_hbm.at[slots[s][0]], slots[s][4], semg[s])

        def wait_gather(s):
            pltpu.make_async_copy(h_hbm.at[pl.ds(0, C)], slots[s][4],
                                  semg[s]).wait()

        def issue_scatter(s):
            pltpu.async_copy(slots[s][4], acc_sh.at[slots[s][2]], semsc[s],
                             add=True)

        def wait_scatter(s):
            pltpu.make_async_copy(slots[s][4], acc_sh.at[pl.ds(0, C)],
                                  semsc[s]).wait()

        def compute_p(s):
            src_v, dst_v, ds_v, p_v, _ = slots[s]
            for b in range(C // NLANE):
                sl = pl.ds(b * NLANE, NLANE)
                d16 = dst_v[sl]
                ds_v[sl] = d16
                av = plsc.load_gather(as_v, [src_v[sl]])
                bv = plsc.load_gather(ad_v, [d16])
                e = av + bv
                e = jnp.maximum(e, e * 0.2)
                p16 = jnp.exp(e - shift)
                p_v[sl] = p16
                plsc.addupdate_scatter(den_v, [d16], p16)

        def scale(s):
            p_v, rows_v = slots[s][3], slots[s][4]

            @pl.loop(0, C)
            def _(j):
                jv = jnp.full((NLANE,), j, jnp.int32)
                pvec = plsc.load_gather(p_v, [jv])
                for q in range(D // NLANE):
                    sl = pl.ds(q * NLANE, NLANE)
                    rows_v[j, sl] = rows_v[j, sl] * pvec

        zv = jnp.zeros((NLANE,), jnp.float32)

        @pl.loop(0, NACC, step=NLANE)
        def _(i):
            den_v[pl.ds(i, NLANE)] = zv

        rows0 = slots[0][4]

        @pl.loop(0, C)
        def _(j):
            for q in range(D // NLANE):
                rows0[j, pl.ds(q * NLANE, NLANE)] = zv

        zbase = sid * ROWS_PER_TILE
        for t in range(ROWS_PER_TILE // C):
            pltpu.sync_copy(rows0, acc_sh.at[pl.ds(zbase + t * C, C)])
        zt = (ROWS_PER_TILE // C) * C
        if ROWS_PER_TILE % C:
            pltpu.sync_copy(rows0.at[pl.ds(0, ROWS_PER_TILE % C)],
                            acc_sh.at[pl.ds(zbase + zt, ROWS_PER_TILE % C)])

        kc = jnp.where(cid == 0, KA, KB)
        chunk0 = jnp.where(cid == 0, sid * KA, NSUB * KA + sid * KB)
        for s in range(NSLOT):
            issue_idx(s, chunk0 + s)
        for s in range(NSLOT):
            wait_idx(s)
            issue_gather(s)
        plsc.subcore_barrier()

        @pl.loop(0, kc // NSLOT - 1)
        def _(i):
            k0 = chunk0 + NSLOT * i
            for s in range(NSLOT):
                compute_p(s)
                wait_gather(s)
                issue_idx(s, k0 + s + NSLOT)
                scale(s)
                issue_scatter(s)
            for s in range(NSLOT):
                wait_scatter(s)
                wait_idx(s)
                issue_gather(s)

        for s in range(NSLOT):
            compute_p(s)
            wait_gather(s)
            scale(s)
            issue_scatter(s)
        for s in range(NSLOT):
            wait_scatter(s)

        plsc.subcore_barrier()
        pltpu.sync_copy(acc_sh.at[pl.ds(zbase, ROWS_PER_TILE)],
                        acc_out.at[cid, pl.ds(zbase, ROWS_PER_TILE)])
        wid = sid * NCORE + cid
        pltpu.sync_copy(den_v, den_out.at[wid])

    return kern(h, src_idx, dst_idx, a_s, a_d, cvec)


def _tc_mlp(acc, den, bias, W1, b1, W2, b2):
    def body(acc_ref, den_ref, bias_ref, w1_ref, b1_ref, w2_ref, b2_ref,
             y_ref):
        g = acc_ref[0] + acc_ref[1]
        d = jnp.sum(den_ref[...], axis=0).reshape(NACC, 1)
        gat = g / d + bias_ref[...][None, :]
        z = jnp.dot(gat, w1_ref[...], preferred_element_type=jnp.float32)
        z = jnp.maximum(z + b1_ref[...][None, :], 0.0)
        y = jnp.dot(z, w2_ref[...], preferred_element_type=jnp.float32)
        y_ref[...] = jax.nn.sigmoid(y + b2_ref[...][None, :])

    return pl.pallas_call(
        body,
        out_shape=jax.ShapeDtypeStruct((NACC, O), jnp.float32),
    )(acc, den, bias, W1, b1, W2, b2)


def kernel(x, edge_index, W, att_src, att_dst, bias, W1, b1, W2, b2):
    h, a_s2, a_d2, cvec = _tc_head(x, W, att_src, att_dst)

    loop = jnp.arange(N, dtype=jnp.int32)
    npad = EPAD - E - N
    src = jnp.concatenate([edge_index[0], loop,
                           jnp.zeros((npad,), jnp.int32)])
    dst = jnp.concatenate([edge_index[1], loop,
                           jnp.full((npad,), DUMMY, jnp.int32)])
    src = src.reshape(TOT_CHUNKS, C)
    dst = dst.reshape(TOT_CHUNKS, C)

    a_s = jnp.pad(a_s2.reshape(-1), (0, PADN - N))
    a_d = jnp.pad(a_d2.reshape(-1), (0, PADN - N))

    acc, den = _sc_gat(h, src, dst, a_s, a_d, cvec.reshape(-1))
    y = _tc_mlp(acc, den, bias, W1, b1, W2, b2)
    return y[:N]


# speedup vs baseline: 1.7418x; 1.0722x over previous
import dataclasses
import functools

import jax
import jax.numpy as jnp
from jax import lax
from jax.experimental import pallas as pl
from jax.experimental.pallas import tpu as pltpu
from jax.experimental.pallas import tpu_sc as plsc

N = 10000
E = 320000
D = 128
H = 256
O = 64

NLANE = 16
NCORE = 2
NSUB = 16
NW = NCORE * NSUB
NSLOT = 4
C = 32
KA = 364
KB = 284
TOT_CHUNKS = NSUB * (KA + KB)
EPAD = TOT_CHUNKS * C
NACC = 10112
ROWS_PER_TILE = NACC // NSUB
PADN = 10016
DUMMY = N


def _tc_head(x, W, att_src, att_dst):
    def body(x_ref, w_ref, as_ref, ad_ref, h_ref, s_ref, d_ref, c_ref):
        h = jnp.dot(x_ref[...], w_ref[...], preferred_element_type=jnp.float32)
        h_ref[...] = h
        a_s = jnp.sum(h * as_ref[...][None, :], axis=1, keepdims=True)
        a_d = jnp.sum(h * ad_ref[...][None, :], axis=1, keepdims=True)
        s_ref[...] = a_s
        d_ref[...] = a_d
        c = jnp.maximum(jnp.max(a_s) + jnp.max(a_d), 0.0)
        c_ref[...] = jnp.full((1, NLANE), c, jnp.float32)

    return pl.pallas_call(
        body,
        out_shape=(
            jax.ShapeDtypeStruct((N, D), jnp.float32),
            jax.ShapeDtypeStruct((N, 1), jnp.float32),
            jax.ShapeDtypeStruct((N, 1), jnp.float32),
            jax.ShapeDtypeStruct((1, NLANE), jnp.float32),
        ),
    )(x, W, att_src, att_dst)


def _sc_gat(h, src_idx, dst_idx, a_s, a_d, cvec):
    mesh = plsc.VectorSubcoreMesh(core_axis_name="c", subcore_axis_name="s")
    cp = pltpu.CompilerParams()
    if "needs_layout_passes" in pltpu.CompilerParams.__dataclass_fields__:
        cp = dataclasses.replace(cp, needs_layout_passes=False)

    scratch = [
        pltpu.VMEM((PADN,), jnp.float32),
        pltpu.VMEM((PADN,), jnp.float32),
        pltpu.VMEM((NLANE,), jnp.float32),
    ]
    for _ in range(NSLOT):
        scratch += [
            pltpu.VMEM((C,), jnp.int32),
            pltpu.VMEM((C,), jnp.int32),
            pltpu.VMEM((C,), jnp.int32),
            pltpu.VMEM((C,), jnp.float32),
            pltpu.VMEM((C, D), jnp.float32),
        ]
    scratch += [
        pltpu.VMEM((NACC,), jnp.float32),
        pltpu.VMEM_SHARED((NACC, D), jnp.float32),
    ]
    scratch += [pltpu.SemaphoreType.DMA] * (3 * NSLOT)

    @functools.partial(
        pl.kernel,
        compiler_params=cp,
        out_type=(
            jax.ShapeDtypeStruct((NCORE, NACC, D), jnp.float32),
            jax.ShapeDtypeStruct((NW, NACC), jnp.float32),
        ),
        mesh=mesh,
        scratch_types=scratch,
    )
    def kern(h_hbm, src_hbm, dst_hbm, as_hbm, ad_hbm, c_hbm, acc_out, den_out,
             *scr):
        as_v, ad_v, c_v = scr[0], scr[1], scr[2]
        slots = [scr[3 + 5 * s: 3 + 5 * (s + 1)] for s in range(NSLOT)]
        den_v = scr[3 + 5 * NSLOT]
        acc_sh = scr[4 + 5 * NSLOT]
        sems = scr[5 + 5 * NSLOT:]
        semi = sems[:NSLOT]
        semg = sems[NSLOT:2 * NSLOT]
        semsc = sems[2 * NSLOT:]

        cid = lax.axis_index("c")
        sid = lax.axis_index("s")

        pltpu.sync_copy(as_hbm, as_v)
        pltpu.sync_copy(ad_hbm, ad_v)
        pltpu.sync_copy(c_hbm, c_v)
        shift = c_v[...]

        def issue_idx(s, ck):
            src_v, dst_v = slots[s][0], slots[s][1]
            pltpu.async_copy(src_hbm.at[ck], src_v, semi[s])
            pltpu.async_copy(dst_hbm.at[ck], dst_v, semi[s])

        def wait_idx(s):
            src_v, dst_v = slots[s][0], slots[s][1]
            pltpu.make_async_copy(src_hbm.at[0], src_v, semi[s]).wait()
            pltpu.make_async_copy(dst_hbm.at[0], dst_v, semi[s]).wait()

        def issue_gather(s):
            pltpu.async_copy(h_hbm.at[slots[s][0]], slots[s][4], semg[s])

        def wait_gather(s):
            pltpu.make_async_copy(h_hbm.at[pl.ds(0, C)], slots[s][4],
                                  semg[s]).wait()

        def issue_scatter(s):
            pltpu.async_copy(slots[s][4], acc_sh.at[slots[s][2]], semsc[s],
                             add=True)

        def wait_scatter(s):
            pltpu.make_async_copy(slots[s][4], acc_sh.at[pl.ds(0, C)],
                                  semsc[s]).wait()

        def compute_p(s):
            src_v, dst_v, ds_v, p_v, _ = slots[s]
            for b in range(C // NLANE):
                sl = pl.ds(b * NLANE, NLANE)
                d16 = dst_v[sl]
                ds_v[sl] = d16
                av = plsc.load_gather(as_v, [src_v[sl]])
                bv = plsc.load_gather(ad_v, [d16])
                e = av + bv
                e = jnp.maximum(e, e * 0.2)
                p16 = jnp.exp(e - shift)
                p_v[sl] = p16
                plsc.addupdate_scatter(den_v, [d16], p16)

        def scale(s):
            p_v, rows_v = slots[s][3], slots[s][4]

            @pl.loop(0, C)
            def _(j):
                jv = jnp.full((NLANE,), j, jnp.int32)
                pvec = plsc.load_gather(p_v, [jv])
                for q in range(D // NLANE):
                    sl = pl.ds(q * NLANE, NLANE)
                    rows_v[j, sl] = rows_v[j, sl] * pvec

        zv = jnp.zeros((NLANE,), jnp.float32)

        @pl.loop(0, NACC, step=NLANE)
        def _(i):
            den_v[pl.ds(i, NLANE)] = zv

        rows0 = slots[0][4]

        @pl.loop(0, C)
        def _(j):
            for q in range(D // NLANE):
                rows0[j, pl.ds(q * NLANE, NLANE)] = zv

        zbase = sid * ROWS_PER_TILE
        for t in range(ROWS_PER_TILE // C):
            pltpu.sync_copy(rows0, acc_sh.at[pl.ds(zbase + t * C, C)])
        zt = (ROWS_PER_TILE // C) * C
        if ROWS_PER_TILE % C:
            pltpu.sync_copy(rows0.at[pl.ds(0, ROWS_PER_TILE % C)],
                            acc_sh.at[pl.ds(zbase + zt, ROWS_PER_TILE % C)])

        kc = jnp.where(cid == 0, KA, KB)
        chunk0 = jnp.where(cid == 0, sid * KA, NSUB * KA + sid * KB)
        for s in range(NSLOT):
            issue_idx(s, chunk0 + s)
        for s in range(NSLOT):
            wait_idx(s)
            issue_gather(s)
        plsc.subcore_barrier()

        @pl.loop(0, kc // NSLOT - 1)
        def _(i):
            k0 = chunk0 + NSLOT * i
            for s in range(NSLOT):
                compute_p(s)
                wait_gather(s)
                issue_idx(s, k0 + s + NSLOT)
                scale(s)
                issue_scatter(s)
            for s in range(NSLOT):
                wait_scatter(s)
                wait_idx(s)
                issue_gather(s)

        for s in range(NSLOT):
            compute_p(s)
            wait_gather(s)
            scale(s)
            issue_scatter(s)
        for s in range(NSLOT):
            wait_scatter(s)

        plsc.subcore_barrier()
        pltpu.sync_copy(acc_sh.at[pl.ds(zbase, ROWS_PER_TILE)],
                        acc_out.at[cid, pl.ds(zbase, ROWS_PER_TILE)])
        wid = sid * NCORE + cid
        pltpu.sync_copy(den_v, den_out.at[wid])

    return kern(h, src_idx, dst_idx, a_s, a_d, cvec)


def _tc_mlp(acc, den, bias, W1, b1, W2, b2):
    def body(acc_ref, den_ref, bias_ref, w1_ref, b1_ref, w2_ref, b2_ref,
             y_ref):
        g = acc_ref[0] + acc_ref[1]
        d = jnp.sum(den_ref[...], axis=0).reshape(NACC, 1)
        gat = g / d + bias_ref[...][None, :]
        z = jnp.dot(gat, w1_ref[...], preferred_element_type=jnp.float32)
        z = jnp.maximum(z + b1_ref[...][None, :], 0.0)
        y = jnp.dot(z, w2_ref[...], preferred_element_type=jnp.float32)
        y_ref[...] = jax.nn.sigmoid(y + b2_ref[...][None, :])

    return pl.pallas_call(
        body,
        out_shape=jax.ShapeDtypeStruct((NACC, O), jnp.float32),
    )(acc, den, bias, W1, b1, W2, b2)


def kernel(x, edge_index, W, att_src, att_dst, bias, W1, b1, W2, b2):
    h, a_s2, a_d2, cvec = _tc_head(x, W, att_src, att_dst)

    loop = jnp.arange(N, dtype=jnp.int32)
    npad = EPAD - E - N
    src = jnp.concatenate([edge_index[0], loop,
                           jnp.zeros((npad,), jnp.int32)])
    dst = jnp.concatenate([edge_index[1], loop,
                           jnp.full((npad,), DUMMY, jnp.int32)])
    src = src.reshape(TOT_CHUNKS, C)
    dst = dst.reshape(TOT_CHUNKS, C)

    a_s = jnp.pad(a_s2.reshape(-1), (0, PADN - N))
    a_d = jnp.pad(a_d2.reshape(-1), (0, PADN - N))

    acc, den = _sc_gat(h, src, dst, a_s, a_d, cvec.reshape(-1))
    y = _tc_mlp(acc, den, bias, W1, b1, W2, b2)
    return y[:N]


# speedup vs baseline: 1.8029x; 1.0350x over previous
import dataclasses
import functools

import jax
import jax.numpy as jnp
from jax import lax
from jax.experimental import pallas as pl
from jax.experimental.pallas import tpu as pltpu
from jax.experimental.pallas import tpu_sc as plsc

N = 10000
E = 320000
D = 128
H = 256
O = 64

NLANE = 16
NCORE = 2
NSUB = 16
NW = NCORE * NSUB
NSLOT = 4
C = 32
KA = 364
KB = 284
TOT_CHUNKS = NSUB * (KA + KB)
EPAD = TOT_CHUNKS * C
NACC = 10112
ROWS_PER_TILE = NACC // NSUB
PADN = 10016
DUMMY = N


def _tc_head(x, W, att_src, att_dst):
    def body(x_ref, w_ref, as_ref, ad_ref, h_ref, s_ref, d_ref, c_ref):
        h = jnp.dot(x_ref[...], w_ref[...], preferred_element_type=jnp.float32)
        h_ref[...] = h
        a_s = jnp.sum(h * as_ref[...][None, :], axis=1, keepdims=True)
        a_d = jnp.sum(h * ad_ref[...][None, :], axis=1, keepdims=True)
        s_ref[...] = a_s
        d_ref[...] = a_d
        c = jnp.maximum(jnp.max(a_s) + jnp.max(a_d), 0.0)
        c_ref[...] = jnp.full((1, NLANE), c, jnp.float32)

    return pl.pallas_call(
        body,
        out_shape=(
            jax.ShapeDtypeStruct((N, D), jnp.float32),
            jax.ShapeDtypeStruct((N, 1), jnp.float32),
            jax.ShapeDtypeStruct((N, 1), jnp.float32),
            jax.ShapeDtypeStruct((1, NLANE), jnp.float32),
        ),
    )(x, W, att_src, att_dst)


def _sc_gat(h, src_idx, dst_idx, a_s, a_d, cvec):
    mesh = plsc.VectorSubcoreMesh(core_axis_name="c", subcore_axis_name="s")
    cp = pltpu.CompilerParams()
    if "needs_layout_passes" in pltpu.CompilerParams.__dataclass_fields__:
        cp = dataclasses.replace(cp, needs_layout_passes=False)

    scratch = [
        pltpu.VMEM((PADN,), jnp.float32),
        pltpu.VMEM((PADN,), jnp.float32),
        pltpu.VMEM((NLANE,), jnp.float32),
    ]
    for _ in range(NSLOT):
        scratch += [
            pltpu.VMEM((C,), jnp.int32),
            pltpu.VMEM((C,), jnp.int32),
            pltpu.VMEM((C,), jnp.int32),
            pltpu.VMEM((C,), jnp.float32),
            pltpu.VMEM((C, D), jnp.float32),
        ]
    scratch += [
        pltpu.VMEM((NACC,), jnp.float32),
        pltpu.VMEM_SHARED((NACC, D), jnp.float32),
    ]
    scratch += [pltpu.SemaphoreType.DMA] * (3 * NSLOT)

    @functools.partial(
        pl.kernel,
        compiler_params=cp,
        out_type=(
            jax.ShapeDtypeStruct((NCORE, NACC, D), jnp.float32),
            jax.ShapeDtypeStruct((NW, NACC), jnp.float32),
        ),
        mesh=mesh,
        scratch_types=scratch,
    )
    def kern(h_hbm, src_hbm, dst_hbm, as_hbm, ad_hbm, c_hbm, acc_out, den_out,
             *scr):
        as_v, ad_v, c_v = scr[0], scr[1], scr[2]
        slots = [scr[3 + 5 * s: 3 + 5 * (s + 1)] for s in range(NSLOT)]
        den_v = scr[3 + 5 * NSLOT]
        acc_sh = scr[4 + 5 * NSLOT]
        sems = scr[5 + 5 * NSLOT:]
        semi = sems[:NSLOT]
        semg = sems[NSLOT:2 * NSLOT]
        semsc = sems[2 * NSLOT:]

        cid = lax.axis_index("c")
        sid = lax.axis_index("s")

        pltpu.async_copy(as_hbm, as_v, semsc[0])
        pltpu.async_copy(ad_hbm, ad_v, semsc[1 % NSLOT])
        pltpu.async_copy(c_hbm, c_v, semsc[2 % NSLOT])

        def issue_idx(s, ck):
            src_v, dst_v = slots[s][0], slots[s][1]
            pltpu.async_copy(src_hbm.at[ck], src_v, semi[s])
            pltpu.async_copy(dst_hbm.at[ck], dst_v, semi[s])

        def wait_idx(s):
            src_v, dst_v = slots[s][0], slots[s][1]
            pltpu.make_async_copy(src_hbm.at[0], src_v, semi[s]).wait()
            pltpu.make_async_copy(dst_hbm.at[0], dst_v, semi[s]).wait()

        def issue_gather(s):
            pltpu.async_copy(h_hbm.at[slots[s][0]], slots[s][4], semg[s])

        def wait_gather(s):
            pltpu.make_async_copy(h_hbm.at[pl.ds(0, C)], slots[s][4],
                                  semg[s]).wait()

        def issue_scatter(s):
            pltpu.async_copy(slots[s][4], acc_sh.at[slots[s][2]], semsc[s],
                             add=True)

        def wait_scatter(s):
            pltpu.make_async_copy(slots[s][4], acc_sh.at[pl.ds(0, C)],
                                  semsc[s]).wait()

        def compute_p(s):
            src_v, dst_v, ds_v, p_v, _ = slots[s]
            for b in range(C // NLANE):
                sl = pl.ds(b * NLANE, NLANE)
                d16 = dst_v[sl]
                ds_v[sl] = d16
                av = plsc.load_gather(as_v, [src_v[sl]])
                bv = plsc.load_gather(ad_v, [d16])
                e = av + bv
                e = jnp.maximum(e, e * 0.2)
                p16 = jnp.exp(e - shift)
                p_v[sl] = p16
                plsc.addupdate_scatter(den_v, [d16], p16)

        def scale(s):
            p_v, rows_v = slots[s][3], slots[s][4]

            @pl.loop(0, C, unroll=4)
            def _(j):
                jv = jnp.full((NLANE,), j, jnp.int32)
                pvec = plsc.load_gather(p_v, [jv])
                for q in range(D // NLANE):
                    sl = pl.ds(q * NLANE, NLANE)
                    rows_v[j, sl] = rows_v[j, sl] * pvec

        zv = jnp.zeros((NLANE,), jnp.float32)

        @pl.loop(0, NACC, step=NLANE)
        def _(i):
            den_v[pl.ds(i, NLANE)] = zv

        rows0 = slots[0][4]

        @pl.loop(0, C)
        def _(j):
            for q in range(D // NLANE):
                rows0[j, pl.ds(q * NLANE, NLANE)] = zv

        zbase = sid * ROWS_PER_TILE
        nz = ROWS_PER_TILE // C
        ztail = ROWS_PER_TILE % C
        for t in range(nz):
            pltpu.async_copy(rows0, acc_sh.at[pl.ds(zbase + t * C, C)],
                             semg[t % NSLOT])
        if ztail:
            pltpu.async_copy(rows0.at[pl.ds(0, ztail)],
                             acc_sh.at[pl.ds(zbase + nz * C, ztail)],
                             semg[nz % NSLOT])
        for t in range(nz):
            pltpu.make_async_copy(rows0, acc_sh.at[pl.ds(0, C)],
                                  semg[t % NSLOT]).wait()
        if ztail:
            pltpu.make_async_copy(rows0.at[pl.ds(0, ztail)],
                                  acc_sh.at[pl.ds(0, ztail)],
                                  semg[nz % NSLOT]).wait()

        kc = jnp.where(cid == 0, KA, KB)
        chunk0 = jnp.where(cid == 0, sid * KA, NSUB * KA + sid * KB)
        for s in range(NSLOT):
            issue_idx(s, chunk0 + s)
        for s in range(NSLOT):
            wait_idx(s)
            issue_gather(s)
        pltpu.make_async_copy(as_hbm, as_v, semsc[0]).wait()
        pltpu.make_async_copy(ad_hbm, ad_v, semsc[1 % NSLOT]).wait()
        pltpu.make_async_copy(c_hbm, c_v, semsc[2 % NSLOT]).wait()
        shift = c_v[...]
        plsc.subcore_barrier()

        @pl.loop(0, kc // NSLOT - 1)
        def _(i):
            k0 = chunk0 + NSLOT * i
            for s in range(NSLOT):
                compute_p(s)
                wait_gather(s)
                issue_idx(s, k0 + s + NSLOT)
                scale(s)
                issue_scatter(s)
            for s in range(NSLOT):
                wait_scatter(s)
                wait_idx(s)
                issue_gather(s)

        for s in range(NSLOT):
            compute_p(s)
            wait_gather(s)
            scale(s)
            issue_scatter(s)
        for s in range(NSLOT):
            wait_scatter(s)

        plsc.subcore_barrier()
        pltpu.sync_copy(acc_sh.at[pl.ds(zbase, ROWS_PER_TILE)],
                        acc_out.at[cid, pl.ds(zbase, ROWS_PER_TILE)])
        wid = sid * NCORE + cid
        pltpu.sync_copy(den_v, den_out.at[wid])

    return kern(h, src_idx, dst_idx, a_s, a_d, cvec)


def _tc_mlp(acc, den, bias, W1, b1, W2, b2):
    def body(acc_ref, den_ref, bias_ref, w1_ref, b1_ref, w2_ref, b2_ref,
             y_ref):
        g = acc_ref[0] + acc_ref[1]
        d = jnp.sum(den_ref[...], axis=0).reshape(NACC, 1)
        gat = g / d + bias_ref[...][None, :]
        z = jnp.dot(gat, w1_ref[...], preferred_element_type=jnp.float32)
        z = jnp.maximum(z + b1_ref[...][None, :], 0.0)
        y = jnp.dot(z, w2_ref[...], preferred_element_type=jnp.float32)
        y_ref[...] = jax.nn.sigmoid(y + b2_ref[...][None, :])

    return pl.pallas_call(
        body,
        out_shape=jax.ShapeDtypeStruct((NACC, O), jnp.float32),
    )(acc, den, bias, W1, b1, W2, b2)


def kernel(x, edge_index, W, att_src, att_dst, bias, W1, b1, W2, b2):
    h, a_s2, a_d2, cvec = _tc_head(x, W, att_src, att_dst)

    loop = jnp.arange(N, dtype=jnp.int32)
    npad = EPAD - E - N
    src = jnp.concatenate([edge_index[0], loop,
                           jnp.zeros((npad,), jnp.int32)])
    dst = jnp.concatenate([edge_index[1], loop,
                           jnp.full((npad,), DUMMY, jnp.int32)])
    src = src.reshape(TOT_CHUNKS, C)
    dst = dst.reshape(TOT_CHUNKS, C)

    a_s = jnp.pad(a_s2.reshape(-1), (0, PADN - N))
    a_d = jnp.pad(a_d2.reshape(-1), (0, PADN - N))

    acc, den = _sc_gat(h, src, dst, a_s, a_d, cvec.reshape(-1))
    y = _tc_mlp(acc, den, bias, W1, b1, W2, b2)
    return y[:N]


# speedup vs baseline: 1.9118x; 1.0604x over previous
import dataclasses
import functools

import jax
import jax.numpy as jnp
from jax import lax
from jax.experimental import pallas as pl
from jax.experimental.pallas import tpu as pltpu
from jax.experimental.pallas import tpu_sc as plsc

N = 10000
E = 320000
D = 128
H = 256
O = 64

NLANE = 16
NCORE = 2
NSUB = 16
NW = NCORE * NSUB
NSLOT = 4
C = 32
E_CHUNKS = E // C
KA = 361
KB = 264
SELF_PER_TILE = 10
SELF_CHUNKS = NW * SELF_PER_TILE
NACC = 10112
ROWS_PER_TILE = NACC // NSUB
PADN = 10016
DUMMY = N
assert NSUB * (KA + KB) == E_CHUNKS


def _tc_head(x, W, att_src, att_dst):
    def body(x_ref, w_ref, as_ref, ad_ref, h_ref, s_ref, d_ref, c_ref):
        h = jnp.dot(x_ref[...], w_ref[...], preferred_element_type=jnp.float32)
        h_ref[...] = h
        zpad = jnp.zeros((PADN - N, 1), jnp.float32)
        a_s = jnp.sum(h * as_ref[...][None, :], axis=1, keepdims=True)
        a_d = jnp.sum(h * ad_ref[...][None, :], axis=1, keepdims=True)
        s_ref[...] = jnp.concatenate([a_s, zpad], axis=0)
        d_ref[...] = jnp.concatenate([a_d, zpad], axis=0)
        c = jnp.maximum(jnp.max(a_s) + jnp.max(a_d), 0.0)
        c_ref[...] = jnp.full((1, NLANE), c, jnp.float32)

    return pl.pallas_call(
        body,
        out_shape=(
            jax.ShapeDtypeStruct((N, D), jnp.float32),
            jax.ShapeDtypeStruct((PADN, 1), jnp.float32),
            jax.ShapeDtypeStruct((PADN, 1), jnp.float32),
            jax.ShapeDtypeStruct((1, NLANE), jnp.float32),
        ),
    )(x, W, att_src, att_dst)


def _sc_gat(h, esrc, edst, ssrc, sdst, a_s, a_d, cvec):
    mesh = plsc.VectorSubcoreMesh(core_axis_name="c", subcore_axis_name="s")
    cp = pltpu.CompilerParams()
    if "needs_layout_passes" in pltpu.CompilerParams.__dataclass_fields__:
        cp = dataclasses.replace(cp, needs_layout_passes=False)

    scratch = [
        pltpu.VMEM((PADN,), jnp.float32),
        pltpu.VMEM((PADN,), jnp.float32),
        pltpu.VMEM((NLANE,), jnp.float32),
    ]
    for _ in range(NSLOT):
        scratch += [
            pltpu.VMEM((C,), jnp.int32),
            pltpu.VMEM((C,), jnp.int32),
            pltpu.VMEM((C,), jnp.int32),
            pltpu.VMEM((C,), jnp.float32),
            pltpu.VMEM((C, D), jnp.float32),
        ]
    scratch += [
        pltpu.VMEM((NACC,), jnp.float32),
        pltpu.VMEM_SHARED((NACC, D), jnp.float32),
    ]
    scratch += [pltpu.SemaphoreType.DMA] * (3 * NSLOT)

    @functools.partial(
        pl.kernel,
        compiler_params=cp,
        out_type=(
            jax.ShapeDtypeStruct((NCORE, NACC, D), jnp.float32),
            jax.ShapeDtypeStruct((NW, NACC), jnp.float32),
        ),
        mesh=mesh,
        scratch_types=scratch,
    )
    def kern(h_hbm, esrc_hbm, edst_hbm, ssrc_hbm, sdst_hbm,
             as_hbm, ad_hbm, c_hbm, acc_out, den_out, *scr):
        as_v, ad_v, c_v = scr[0], scr[1], scr[2]
        slots = [scr[3 + 5 * s: 3 + 5 * (s + 1)] for s in range(NSLOT)]
        den_v = scr[3 + 5 * NSLOT]
        acc_sh = scr[4 + 5 * NSLOT]
        sems = scr[5 + 5 * NSLOT:]
        semi = sems[:NSLOT]
        semg = sems[NSLOT:2 * NSLOT]
        semsc = sems[2 * NSLOT:]

        cid = lax.axis_index("c")
        sid = lax.axis_index("s")

        pltpu.async_copy(as_hbm, as_v, semsc[0])
        pltpu.async_copy(ad_hbm, ad_v, semsc[1 % NSLOT])
        pltpu.async_copy(c_hbm, c_v, semsc[2 % NSLOT])

        def issue_idx(s, ck, sh, dh):
            src_v, dst_v = slots[s][0], slots[s][1]
            pltpu.async_copy(sh.at[ck], src_v, semi[s])
            pltpu.async_copy(dh.at[ck], dst_v, semi[s])

        def wait_idx(s):
            src_v, dst_v = slots[s][0], slots[s][1]
            pltpu.make_async_copy(esrc_hbm.at[0], src_v, semi[s]).wait()
            pltpu.make_async_copy(edst_hbm.at[0], dst_v, semi[s]).wait()

        def issue_gather(s):
            pltpu.async_copy(h_hbm.at[slots[s][0]], slots[s][4], semg[s])

        def wait_gather(s):
            pltpu.make_async_copy(h_hbm.at[pl.ds(0, C)], slots[s][4],
                                  semg[s]).wait()

        def issue_scatter(s):
            pltpu.async_copy(slots[s][4], acc_sh.at[slots[s][2]], semsc[s],
                             add=True)

        def wait_scatter(s):
            pltpu.make_async_copy(slots[s][4], acc_sh.at[pl.ds(0, C)],
                                  semsc[s]).wait()

        def compute_p(s):
            src_v, dst_v, ds_v, p_v, _ = slots[s]
            shift = c_v[...]
            for b in range(C // NLANE):
                sl = pl.ds(b * NLANE, NLANE)
                d16 = dst_v[sl]
                ds_v[sl] = d16
                av = plsc.load_gather(as_v, [src_v[sl]])
                bv = plsc.load_gather(ad_v, [d16])
                e = av + bv
                e = jnp.maximum(e, e * 0.2)
                p16 = jnp.exp(e - shift)
                p_v[sl] = p16
                plsc.addupdate_scatter(den_v, [d16], p16)

        def scale(s):
            p_v, rows_v = slots[s][3], slots[s][4]

            @pl.loop(0, C, unroll=4)
            def _(j):
                jv = jnp.full((NLANE,), j, jnp.int32)
                pvec = plsc.load_gather(p_v, [jv])
                for q in range(D // NLANE):
                    sl = pl.ds(q * NLANE, NLANE)
                    rows_v[j, sl] = rows_v[j, sl] * pvec

        zv = jnp.zeros((NLANE,), jnp.float32)

        @pl.loop(0, NACC, step=NLANE)
        def _(i):
            den_v[pl.ds(i, NLANE)] = zv

        rows0 = slots[0][4]

        @pl.loop(0, C)
        def _(j):
            for q in range(D // NLANE):
                rows0[j, pl.ds(q * NLANE, NLANE)] = zv

        zbase = sid * ROWS_PER_TILE
        nz = ROWS_PER_TILE // C
        ztail = ROWS_PER_TILE % C
        for t in range(nz):
            pltpu.async_copy(rows0, acc_sh.at[pl.ds(zbase + t * C, C)],
                             semg[t % NSLOT])
        if ztail:
            pltpu.async_copy(rows0.at[pl.ds(0, ztail)],
                             acc_sh.at[pl.ds(zbase + nz * C, ztail)],
                             semg[nz % NSLOT])
        for t in range(nz):
            pltpu.make_async_copy(rows0, acc_sh.at[pl.ds(0, C)],
                                  semg[t % NSLOT]).wait()
        if ztail:
            pltpu.make_async_copy(rows0.at[pl.ds(0, ztail)],
                                  acc_sh.at[pl.ds(0, ztail)],
                                  semg[nz % NSLOT]).wait()

        def pipeline(sh, dh, chunk0, kc, barrier_after_prime):
            for s in range(NSLOT):
                issue_idx(s, chunk0 + s, sh, dh)
            for s in range(NSLOT):
                wait_idx(s)
                issue_gather(s)
            if barrier_after_prime:
                pltpu.make_async_copy(as_hbm, as_v, semsc[0]).wait()
                pltpu.make_async_copy(ad_hbm, ad_v, semsc[1 % NSLOT]).wait()
                pltpu.make_async_copy(c_hbm, c_v, semsc[2 % NSLOT]).wait()
                plsc.subcore_barrier()

            @pl.loop(0, kc // NSLOT - 1)
            def _(i):
                k0 = chunk0 + NSLOT * i
                for s in range(NSLOT):
                    compute_p(s)
                    wait_gather(s)
                    issue_idx(s, k0 + s + NSLOT, sh, dh)
                    scale(s)
                    issue_scatter(s)
                for s in range(NSLOT):
                    wait_scatter(s)
                    wait_idx(s)
                    issue_gather(s)

            for s in range(NSLOT):
                compute_p(s)
                wait_gather(s)
                scale(s)
                issue_scatter(s)
            for s in range(NSLOT):
                wait_scatter(s)

            @pl.loop(0, kc - (kc // NSLOT) * NSLOT)
            def _(r):
                ck = chunk0 + (kc // NSLOT) * NSLOT + r
                issue_idx(0, ck, sh, dh)
                wait_idx(0)
                issue_gather(0)
                compute_p(0)
                wait_gather(0)
                scale(0)
                issue_scatter(0)
                wait_scatter(0)

        kc = jnp.where(cid == 0, KA, KB)
        chunk0 = jnp.where(cid == 0, sid * KA, NSUB * KA + sid * KB)
        wid = sid * NCORE + cid
        pipeline(esrc_hbm, edst_hbm, chunk0, kc, True)
        pipeline(ssrc_hbm, sdst_hbm, wid * SELF_PER_TILE, SELF_PER_TILE,
                 False)

        plsc.subcore_barrier()
        pltpu.sync_copy(acc_sh.at[pl.ds(zbase, ROWS_PER_TILE)],
                        acc_out.at[cid, pl.ds(zbase, ROWS_PER_TILE)])
        pltpu.sync_copy(den_v, den_out.at[wid])

    return kern(h, esrc, edst, ssrc, sdst, a_s, a_d, cvec)


def _tc_mlp(acc, den, bias, W1, b1, W2, b2):
    def body(acc_ref, den_ref, bias_ref, w1_ref, b1_ref, w2_ref, b2_ref,
             y_ref):
        g = acc_ref[0] + acc_ref[1]
        d = jnp.sum(den_ref[...], axis=0).reshape(NACC, 1)
        gat = g / d + bias_ref[...][None, :]
        z = jnp.dot(gat, w1_ref[...], preferred_element_type=jnp.float32)
        z = jnp.maximum(z + b1_ref[...][None, :], 0.0)
        y = jnp.dot(z, w2_ref[...], preferred_element_type=jnp.float32)
        y_ref[...] = jax.nn.sigmoid(y + b2_ref[...][None, :])

    return pl.pallas_call(
        body,
        out_shape=jax.ShapeDtypeStruct((NACC, O), jnp.float32),
    )(acc, den, bias, W1, b1, W2, b2)


def kernel(x, edge_index, W, att_src, att_dst, bias, W1, b1, W2, b2):
    h, a_s2, a_d2, cvec = _tc_head(x, W, att_src, att_dst)

    esrc = edge_index[0].reshape(E_CHUNKS, C)
    edst = edge_index[1].reshape(E_CHUNKS, C)
    node = jnp.arange(SELF_CHUNKS * C, dtype=jnp.int32)
    ssrc = jnp.where(node < N, node, 0).reshape(SELF_CHUNKS, C)
    sdst = jnp.where(node < N, node, DUMMY).reshape(SELF_CHUNKS, C)

    acc, den = _sc_gat(h, esrc, edst, ssrc, sdst,
                       a_s2.reshape(-1), a_d2.reshape(-1), cvec.reshape(-1))
    y = _tc_mlp(acc, den, bias, W1, b1, W2, b2)
    return y[:N]


# speedup vs baseline: 2.2209x; 1.1616x over previous
import dataclasses
import functools

import jax
import jax.numpy as jnp
from jax import lax
from jax.experimental import pallas as pl
from jax.experimental.pallas import tpu as pltpu
from jax.experimental.pallas import tpu_sc as plsc

N = 10000
E = 320000
D = 128
H = 256
O = 64

NLANE = 16
NCORE = 2
NSUB = 16
NW = NCORE * NSUB
NSLOT = 4
C = 32
E_CHUNKS = E // C
KA = 313
KB = 312
SELF_PER_TILE = 10
SELF_CHUNKS = NW * SELF_PER_TILE
NACC = 10112
ROWS_PER_TILE = NACC // NSUB
PADN = 10016
DUMMY = N
assert NSUB * (KA + KB) == E_CHUNKS


def _tc_head(x, W, att_src, att_dst):
    def body(x_ref, w_ref, as_ref, ad_ref, h_ref, s_ref, d_ref, c_ref):
        h = jnp.dot(x_ref[...], w_ref[...], preferred_element_type=jnp.float32)
        h_ref[...] = h
        zpad = jnp.zeros((PADN - N,), jnp.float32)
        a_s = jnp.sum(h * as_ref[...][None, :], axis=1)
        a_d = jnp.sum(h * ad_ref[...][None, :], axis=1)
        s_ref[...] = jnp.concatenate([a_s, zpad])
        d_ref[...] = jnp.concatenate([a_d, zpad])
        c = jnp.maximum(jnp.max(a_s) + jnp.max(a_d), 0.0)
        c_ref[...] = jnp.full((NLANE,), c, jnp.float32)

    return pl.pallas_call(
        body,
        out_shape=(
            jax.ShapeDtypeStruct((N, D), jnp.float32),
            jax.ShapeDtypeStruct((PADN,), jnp.float32),
            jax.ShapeDtypeStruct((PADN,), jnp.float32),
            jax.ShapeDtypeStruct((NLANE,), jnp.float32),
        ),
    )(x, W, att_src, att_dst)


def _sc_gat(h, eidx, ssrc, sdst, a_s, a_d, cvec):
    mesh = plsc.VectorSubcoreMesh(core_axis_name="c", subcore_axis_name="s")
    cp = pltpu.CompilerParams()
    if "needs_layout_passes" in pltpu.CompilerParams.__dataclass_fields__:
        cp = dataclasses.replace(cp, needs_layout_passes=False)

    scratch = [
        pltpu.VMEM((PADN,), jnp.float32),
        pltpu.VMEM((PADN,), jnp.float32),
        pltpu.VMEM((NLANE,), jnp.float32),
    ]
    for _ in range(NSLOT):
        scratch += [
            pltpu.VMEM((C,), jnp.int32),
            pltpu.VMEM((C,), jnp.int32),
            pltpu.VMEM((C,), jnp.int32),
            pltpu.VMEM((C,), jnp.float32),
            pltpu.VMEM((C, D), jnp.float32),
        ]
    scratch += [
        pltpu.VMEM((NACC,), jnp.float32),
        pltpu.VMEM_SHARED((NACC, D), jnp.float32),
    ]
    scratch += [pltpu.SemaphoreType.DMA] * (3 * NSLOT)

    @functools.partial(
        pl.kernel,
        compiler_params=cp,
        out_type=(
            jax.ShapeDtypeStruct((NCORE, NACC, D), jnp.float32),
            jax.ShapeDtypeStruct((NW, NACC), jnp.float32),
        ),
        mesh=mesh,
        scratch_types=scratch,
    )
    def kern(h_hbm, eidx_hbm, ssrc_hbm, sdst_hbm,
             as_hbm, ad_hbm, c_hbm, acc_out, den_out, *scr):
        as_v, ad_v, c_v = scr[0], scr[1], scr[2]
        slots = [scr[3 + 5 * s: 3 + 5 * (s + 1)] for s in range(NSLOT)]
        den_v = scr[3 + 5 * NSLOT]
        acc_sh = scr[4 + 5 * NSLOT]
        sems = scr[5 + 5 * NSLOT:]
        semi = sems[:NSLOT]
        semg = sems[NSLOT:2 * NSLOT]
        semsc = sems[2 * NSLOT:]

        cid = lax.axis_index("c")
        sid = lax.axis_index("s")

        pltpu.async_copy(as_hbm, as_v, semsc[0])
        pltpu.async_copy(ad_hbm, ad_v, semsc[1 % NSLOT])
        pltpu.async_copy(c_hbm, c_v, semsc[2 % NSLOT])

        def issue_idx(s, ck, get_src, get_dst):
            src_v, dst_v = slots[s][0], slots[s][1]
            pltpu.async_copy(get_src(ck), src_v, semi[s])
            pltpu.async_copy(get_dst(ck), dst_v, semi[s])

        def wait_idx(s):
            src_v, dst_v = slots[s][0], slots[s][1]
            pltpu.make_async_copy(ssrc_hbm.at[0], src_v, semi[s]).wait()
            pltpu.make_async_copy(ssrc_hbm.at[0], dst_v, semi[s]).wait()

        def issue_gather(s):
            pltpu.async_copy(h_hbm.at[slots[s][0]], slots[s][4], semg[s])

        def wait_gather(s):
            pltpu.make_async_copy(h_hbm.at[pl.ds(0, C)], slots[s][4],
                                  semg[s]).wait()

        def issue_scatter(s):
            pltpu.async_copy(slots[s][4], acc_sh.at[slots[s][2]], semsc[s],
                             add=True)

        def wait_scatter(s):
            pltpu.make_async_copy(slots[s][4], acc_sh.at[pl.ds(0, C)],
                                  semsc[s]).wait()

        def compute_p(s):
            src_v, dst_v, ds_v, p_v, _ = slots[s]
            shift = c_v[...]
            for b in range(C // NLANE):
                sl = pl.ds(b * NLANE, NLANE)
                d16 = dst_v[sl]
                ds_v[sl] = d16
                av = plsc.load_gather(as_v, [src_v[sl]])
                bv = plsc.load_gather(ad_v, [d16])
                e = av + bv
                e = jnp.maximum(e, e * 0.2)
                p16 = jnp.exp(e - shift)
                p_v[sl] = p16
                plsc.addupdate_scatter(den_v, [d16], p16)

        def scale(s):
            p_v, rows_v = slots[s][3], slots[s][4]

            @pl.loop(0, C, unroll=4)
            def _(j):
                jv = jnp.full((NLANE,), j, jnp.int32)
                pvec = plsc.load_gather(p_v, [jv])
                for q in range(D // NLANE):
                    sl = pl.ds(q * NLANE, NLANE)
                    rows_v[j, sl] = rows_v[j, sl] * pvec

        zv = jnp.zeros((NLANE,), jnp.float32)

        @pl.loop(0, NACC, step=NLANE)
        def _(i):
            den_v[pl.ds(i, NLANE)] = zv

        rows0 = slots[0][4]

        @pl.loop(0, C)
        def _(j):
            for q in range(D // NLANE):
                rows0[j, pl.ds(q * NLANE, NLANE)] = zv

        zbase = sid * ROWS_PER_TILE
        nz = ROWS_PER_TILE // C
        ztail = ROWS_PER_TILE % C
        for t in range(nz):
            pltpu.async_copy(rows0, acc_sh.at[pl.ds(zbase + t * C, C)],
                             semg[t % NSLOT])
        if ztail:
            pltpu.async_copy(rows0.at[pl.ds(0, ztail)],
                             acc_sh.at[pl.ds(zbase + nz * C, ztail)],
                             semg[nz % NSLOT])
        for t in range(nz):
            pltpu.make_async_copy(rows0, acc_sh.at[pl.ds(0, C)],
                                  semg[t % NSLOT]).wait()
        if ztail:
            pltpu.make_async_copy(rows0.at[pl.ds(0, ztail)],
                                  acc_sh.at[pl.ds(0, ztail)],
                                  semg[nz % NSLOT]).wait()

        def pipeline(get_src, get_dst, chunk0, kc, barrier_after_prime):
            for s in range(NSLOT):
                issue_idx(s, chunk0 + s, get_src, get_dst)
            for s in range(NSLOT):
                wait_idx(s)
                issue_gather(s)
            if barrier_after_prime:
                pltpu.make_async_copy(as_hbm, as_v, semsc[0]).wait()
                pltpu.make_async_copy(ad_hbm, ad_v, semsc[1 % NSLOT]).wait()
                pltpu.make_async_copy(c_hbm, c_v, semsc[2 % NSLOT]).wait()
                plsc.subcore_barrier()

            @pl.loop(0, kc // NSLOT - 1)
            def _(i):
                k0 = chunk0 + NSLOT * i
                for s in range(NSLOT):
                    compute_p(s)
                    wait_gather(s)
                    issue_idx(s, k0 + s + NSLOT, get_src, get_dst)
                    scale(s)
                    issue_scatter(s)
                for s in range(NSLOT):
                    wait_scatter(s)
                    wait_idx(s)
                    issue_gather(s)

            for s in range(NSLOT):
                compute_p(s)
                wait_gather(s)
                scale(s)
                issue_scatter(s)
            for s in range(NSLOT):
                wait_scatter(s)

            @pl.loop(0, kc - (kc // NSLOT) * NSLOT)
            def _(r):
                ck = chunk0 + (kc // NSLOT) * NSLOT + r
                issue_idx(0, ck, get_src, get_dst)
                wait_idx(0)
                issue_gather(0)
                compute_p(0)
                wait_gather(0)
                scale(0)
                issue_scatter(0)
                wait_scatter(0)

        kc = jnp.where(cid == 0, KA, KB)
        chunk0 = jnp.where(cid == 0, sid * KA, NSUB * KA + sid * KB)
        wid = sid * NCORE + cid
        pipeline(lambda ck: eidx_hbm.at[0, ck],
                 lambda ck: eidx_hbm.at[1, ck],
                 chunk0, kc, True)
        pipeline(lambda ck: ssrc_hbm.at[ck],
                 lambda ck: sdst_hbm.at[ck],
                 wid * SELF_PER_TILE, SELF_PER_TILE, False)

        plsc.subcore_barrier()
        pltpu.sync_copy(acc_sh.at[pl.ds(zbase, ROWS_PER_TILE)],
                        acc_out.at[cid, pl.ds(zbase, ROWS_PER_TILE)])
        pltpu.sync_copy(den_v, den_out.at[wid])

    return kern(h, eidx, ssrc, sdst, a_s, a_d, cvec)


def _tc_mlp(acc, den, bias, W1, b1, W2, b2):
    def body(acc_ref, den_ref, bias_ref, w1_ref, b1_ref, w2_ref, b2_ref,
             y_ref):
        g = acc_ref[0] + acc_ref[1]
        d = jnp.sum(den_ref[...], axis=0).reshape(NACC, 1)
        gat = g / d + bias_ref[...][None, :]
        z = jnp.dot(gat, w1_ref[...], preferred_element_type=jnp.float32)
        z = jnp.maximum(z + b1_ref[...][None, :], 0.0)
        y = jnp.dot(z, w2_ref[...], preferred_element_type=jnp.float32)
        y_ref[...] = jax.nn.sigmoid(y + b2_ref[...][None, :])

    return pl.pallas_call(
        body,
        out_shape=jax.ShapeDtypeStruct((NACC, O), jnp.float32),
    )(acc, den, bias, W1, b1, W2, b2)


def kernel(x, edge_index, W, att_src, att_dst, bias, W1, b1, W2, b2):
    h, a_s2, a_d2, cvec = _tc_head(x, W, att_src, att_dst)

    node = jnp.arange(SELF_CHUNKS * C, dtype=jnp.int32)
    ssrc = jnp.where(node < N, node, 0).reshape(SELF_CHUNKS, C)
    sdst = jnp.where(node < N, node, DUMMY).reshape(SELF_CHUNKS, C)

    eidx3 = edge_index.reshape(2, E_CHUNKS, C)
    acc, den = _sc_gat(h, eidx3, ssrc, sdst, a_s2, a_d2, cvec)
    y = _tc_mlp(acc, den, bias, W1, b1, W2, b2)
    return y[:N]
